# baseline (speedup 1.0000x reference)
import jax
import jax.numpy as jnp
import numpy as np
from jax.experimental import pallas as pl
from jax.experimental.pallas import tpu as pltpu

BS = 8; NPTS = 2048; NA = 12
DIM_IN = 32; DIM_OUT = 32; KS = 3
APERTURE = 1.6; SIGMA = 0.2; ANN = 3

ROWS_IN = DIM_IN * NA
ROWS_OUT = DIM_OUT * NA
PT = 1024
BB = 2


def _anchor_mix_matrix():
    phi = (1.0 + np.sqrt(5.0)) / 2.0
    verts = []
    for s1 in (-1.0, 1.0):
        for s2 in (-1.0, 1.0):
            verts.append([0.0, s1, s2 * phi])
            verts.append([s1, s2 * phi, 0.0])
            verts.append([s2 * phi, 0.0, s1])
    v = np.asarray(verts, dtype=np.float32)
    anchors = (v / np.linalg.norm(v, axis=1, keepdims=True))[:NA]
    kernels = np.linspace(0.0, APERTURE, KS, dtype=np.float32)
    dots = np.clip(anchors @ anchors.T, -1.0, 1.0).astype(np.float32)
    dists = np.arccos(dots).astype(np.float32)
    diff = dists[:, None, :] - kernels[None, :, None]
    w = np.exp(-(diff.astype(np.float32) ** 2) / np.float32(2.0 * SIGMA))
    w = np.where(dists[:, None, :] <= APERTURE + 1e-6, w.astype(np.float32),
                 np.float32(0.0))
    idx = np.argsort(-w, axis=-1, kind='stable')[..., :ANN]
    topw = np.take_along_axis(w, idx, axis=-1).astype(np.float32)
    topw = (topw / (topw.sum(-1, keepdims=True) + np.float32(1e-9)))
    M = np.zeros((KS, NA, NA), np.float32)
    o_i, k_i, _ = np.meshgrid(np.arange(NA), np.arange(KS), np.arange(ANN),
                              indexing='ij')
    np.add.at(M, (k_i, o_i, idx), topw.astype(np.float32))
    return M


_M_CONST = _anchor_mix_matrix()

_Q_CONST = np.zeros((KS, DIM_IN * KS, ROWS_IN), np.float32)
for _k in range(KS):
    for _j in range(NA):
        for _c in range(DIM_IN):
            _Q_CONST[_k, _c * KS + _k, _j * DIM_IN + _c] = 1.0
_S_CONST = np.ascontiguousarray(
    np.broadcast_to(
        _M_CONST[:, :, None, :, None],
        (KS, NA, DIM_OUT, NA, DIM_IN)).reshape(KS, ROWS_OUT, ROWS_IN))


def _zpconv_kernel(w_ref, b_ref, q_ref, s_ref, x_ref, o_ref, wm_ref):
    @pl.when((pl.program_id(0) == 0) & (pl.program_id(1) == 0))
    def _fold():
        wt = jnp.broadcast_to(w_ref[...][None], (NA, DIM_OUT, DIM_IN * KS))
        wt = wt.reshape(ROWS_OUT, DIM_IN * KS)
        wm = jnp.zeros((ROWS_OUT, ROWS_IN), jnp.float32)
        for k in range(KS):
            wm = wm + s_ref[k] * jax.lax.dot_general(
                wt, q_ref[k], (((1,), (0,)), ((), ())),
                preferred_element_type=jnp.float32)
        wm_ref[...] = wm

    bcol = jnp.broadcast_to(b_ref[0][None], (NA, DIM_OUT, 1))
    bcol = bcol.reshape(ROWS_OUT, 1)
    for i in range(BB):
        acc = jax.lax.dot_general(
            wm_ref[...], x_ref[i], (((1,), (0,)), ((), ())),
            preferred_element_type=jnp.float32)
        o_ref[i] = acc + bcol


@jax.jit
def kernel(xyz, feats, W, bias):
    del xyz
    ft = feats.transpose(0, 3, 1, 2).reshape(BS, ROWS_IN, NPTS)

    n_pt = NPTS // PT
    out = pl.pallas_call(
        _zpconv_kernel,
        grid=(BS // BB, n_pt),
        in_specs=[
            pl.BlockSpec((DIM_OUT, DIM_IN * KS), lambda b, p: (0, 0)),
            pl.BlockSpec((1, DIM_OUT, 1), lambda b, p: (0, 0, 0)),
            pl.BlockSpec((KS, DIM_IN * KS, ROWS_IN), lambda b, p: (0, 0, 0)),
            pl.BlockSpec((KS, ROWS_OUT, ROWS_IN), lambda b, p: (0, 0, 0)),
            pl.BlockSpec((BB, ROWS_IN, PT), lambda b, p: (b, 0, p)),
        ],
        out_specs=pl.BlockSpec((BB, ROWS_OUT, PT), lambda b, p: (b, 0, p)),
        out_shape=jax.ShapeDtypeStruct((BS, ROWS_OUT, NPTS), jnp.float32),
        scratch_shapes=[pltpu.VMEM((ROWS_OUT, ROWS_IN), jnp.float32)],
        compiler_params=pltpu.CompilerParams(
            dimension_semantics=("arbitrary", "arbitrary")),
    )(W, bias, jnp.asarray(_Q_CONST), jnp.asarray(_S_CONST), ft)

    return out.reshape(BS, NA, DIM_OUT, NPTS).transpose(0, 2, 3, 1)


# speedup vs baseline: 1.0445x; 1.0445x over previous
import jax
import jax.numpy as jnp
import numpy as np
from jax.experimental import pallas as pl
from jax.experimental.pallas import tpu as pltpu

BS = 8; NPTS = 2048; NA = 12
DIM_IN = 32; DIM_OUT = 32; KS = 3
APERTURE = 1.6; SIGMA = 0.2; ANN = 3

ROWS_IN = DIM_IN * NA
ROWS_OUT = DIM_OUT * NA
PT = 2048
BB = 2


def _anchor_mix_matrix():
    phi = (1.0 + np.sqrt(5.0)) / 2.0
    verts = []
    for s1 in (-1.0, 1.0):
        for s2 in (-1.0, 1.0):
            verts.append([0.0, s1, s2 * phi])
            verts.append([s1, s2 * phi, 0.0])
            verts.append([s2 * phi, 0.0, s1])
    v = np.asarray(verts, dtype=np.float32)
    anchors = (v / np.linalg.norm(v, axis=1, keepdims=True))[:NA]
    kernels = np.linspace(0.0, APERTURE, KS, dtype=np.float32)
    dots = np.clip(anchors @ anchors.T, -1.0, 1.0).astype(np.float32)
    dists = np.arccos(dots).astype(np.float32)
    diff = dists[:, None, :] - kernels[None, :, None]
    w = np.exp(-(diff.astype(np.float32) ** 2) / np.float32(2.0 * SIGMA))
    w = np.where(dists[:, None, :] <= APERTURE + 1e-6, w.astype(np.float32),
                 np.float32(0.0))
    idx = np.argsort(-w, axis=-1, kind='stable')[..., :ANN]
    topw = np.take_along_axis(w, idx, axis=-1).astype(np.float32)
    topw = (topw / (topw.sum(-1, keepdims=True) + np.float32(1e-9)))
    M = np.zeros((KS, NA, NA), np.float32)
    o_i, k_i, _ = np.meshgrid(np.arange(NA), np.arange(KS), np.arange(ANN),
                              indexing='ij')
    np.add.at(M, (k_i, o_i, idx), topw.astype(np.float32))
    return M


_M_CONST = _anchor_mix_matrix()

_Q_CONST = np.zeros((KS, DIM_IN * KS, ROWS_IN), np.float32)
for _k in range(KS):
    for _j in range(NA):
        for _c in range(DIM_IN):
            _Q_CONST[_k, _c * KS + _k, _j * DIM_IN + _c] = 1.0
_S_CONST = np.ascontiguousarray(
    np.broadcast_to(
        _M_CONST[:, :, None, :, None],
        (KS, NA, DIM_OUT, NA, DIM_IN)).reshape(KS, ROWS_OUT, ROWS_IN))


def _zpconv_kernel(w_ref, b_ref, q_ref, s_ref, x_ref, o_ref, wm_ref):
    @pl.when((pl.program_id(0) == 0) & (pl.program_id(1) == 0))
    def _fold():
        wt = jnp.broadcast_to(w_ref[...][None], (NA, DIM_OUT, DIM_IN * KS))
        wt = wt.reshape(ROWS_OUT, DIM_IN * KS)
        wm = jnp.zeros((ROWS_OUT, ROWS_IN), jnp.float32)
        for k in range(KS):
            wm = wm + s_ref[k] * jax.lax.dot_general(
                wt, q_ref[k], (((1,), (0,)), ((), ())),
                preferred_element_type=jnp.float32)
        wm_ref[...] = wm

    bcol = jnp.broadcast_to(b_ref[0][None], (NA, DIM_OUT, 1))
    bcol = bcol.reshape(ROWS_OUT, 1)
    for i in range(BB):
        acc = jax.lax.dot_general(
            wm_ref[...], x_ref[i], (((1,), (0,)), ((), ())),
            preferred_element_type=jnp.float32)
        o_ref[i] = acc + bcol


@jax.jit
def kernel(xyz, feats, W, bias):
    del xyz
    ft = feats.transpose(0, 3, 1, 2).reshape(BS, ROWS_IN, NPTS)

    n_pt = NPTS // PT
    out = pl.pallas_call(
        _zpconv_kernel,
        grid=(BS // BB, n_pt),
        in_specs=[
            pl.BlockSpec((DIM_OUT, DIM_IN * KS), lambda b, p: (0, 0)),
            pl.BlockSpec((1, DIM_OUT, 1), lambda b, p: (0, 0, 0)),
            pl.BlockSpec((KS, DIM_IN * KS, ROWS_IN), lambda b, p: (0, 0, 0)),
            pl.BlockSpec((KS, ROWS_OUT, ROWS_IN), lambda b, p: (0, 0, 0)),
            pl.BlockSpec((BB, ROWS_IN, PT), lambda b, p: (b, 0, p)),
        ],
        out_specs=pl.BlockSpec((BB, ROWS_OUT, PT), lambda b, p: (b, 0, p)),
        out_shape=jax.ShapeDtypeStruct((BS, ROWS_OUT, NPTS), jnp.float32),
        scratch_shapes=[pltpu.VMEM((ROWS_OUT, ROWS_IN), jnp.float32)],
        compiler_params=pltpu.CompilerParams(
            dimension_semantics=("arbitrary", "arbitrary")),
    )(W, bias, jnp.asarray(_Q_CONST), jnp.asarray(_S_CONST), ft)

    return out.reshape(BS, NA, DIM_OUT, NPTS).transpose(0, 2, 3, 1)
